# baseline (speedup 1.0000x reference)
import jax
import jax.numpy as jnp
from jax.experimental import pallas as pl
from jax.experimental.pallas import tpu as pltpu

_MAX_ITER = 10
_TAU = 1.0
_CHUNK = 128


def _sinkhorn_kernel(t_hbm, out_hbm, k_ref, s_ref, in_sem, out_sem):
    b = pl.program_id(0)
    n = k_ref.shape[0]
    n_chunks = n // _CHUNK

    load = pltpu.make_async_copy(t_hbm.at[b], k_ref, in_sem)
    load.start()
    load.wait()

    def rows(r):
        return pl.ds(r * _CHUNK, _CHUNK)

    def init_chunk(r, _):
        e = jnp.exp(k_ref[rows(r), :] * (1.0 / _TAU))
        k_ref[rows(r), :] = e
        s_ref[rows(r), :] = jnp.sum(e, axis=1, keepdims=True)
        return 0

    jax.lax.fori_loop(0, n_chunks, init_chunk, 0)

    def col_pass():
        def col_chunk(r, acc):
            u = 1.0 / s_ref[rows(r), :]
            return acc + jnp.sum(k_ref[rows(r), :] * u, axis=0, keepdims=True)

        acc0 = jnp.zeros((1, n), dtype=jnp.float32)
        return 1.0 / jax.lax.fori_loop(0, n_chunks, col_chunk, acc0)

    def row_pass(v):
        def row_chunk(r, _):
            s_ref[rows(r), :] = jnp.sum(
                k_ref[rows(r), :] * v, axis=1, keepdims=True
            )
            return 0

        jax.lax.fori_loop(0, n_chunks, row_chunk, 0)

    def iter_body(i, v_unused):
        v = col_pass()
        row_pass(v)
        return v

    jax.lax.fori_loop(0, _MAX_ITER - 1, iter_body, jnp.zeros((1, n), jnp.float32))
    v = col_pass()

    def prod_chunk(r, _):
        u = 1.0 / s_ref[rows(r), :]
        k_ref[rows(r), :] = k_ref[rows(r), :] * u * v
        return 0

    jax.lax.fori_loop(0, n_chunks, prod_chunk, 0)

    store = pltpu.make_async_copy(k_ref, out_hbm.at[b], out_sem)
    store.start()
    store.wait()


@jax.jit
def kernel(t):
    b, n, m = t.shape
    return pl.pallas_call(
        _sinkhorn_kernel,
        grid=(b,),
        in_specs=[pl.BlockSpec(memory_space=pltpu.MemorySpace.HBM)],
        out_specs=pl.BlockSpec(memory_space=pltpu.MemorySpace.HBM),
        out_shape=jax.ShapeDtypeStruct((b, n, m), jnp.float32),
        scratch_shapes=[
            pltpu.VMEM((n, m), jnp.float32),
            pltpu.VMEM((n, 1), jnp.float32),
            pltpu.SemaphoreType.DMA,
            pltpu.SemaphoreType.DMA,
        ],
    )(t)


# speedup vs baseline: 1.4457x; 1.4457x over previous
import jax
import jax.numpy as jnp
from jax.experimental import pallas as pl
from jax.experimental.pallas import tpu as pltpu

_MAX_ITER = 10
_TAU = 1.0
_CHUNK = 128


def _sinkhorn_kernel(t_hbm, out_hbm, k_ref, s_ref, in_sem, out_sem):
    b = pl.program_id(0)
    n = k_ref.shape[0]
    n_chunks = n // _CHUNK

    load = pltpu.make_async_copy(t_hbm.at[b], k_ref, in_sem)
    load.start()
    load.wait()

    def rows(r):
        return pl.ds(r * _CHUNK, _CHUNK)

    def init_chunk(r, acc):
        e = jnp.exp(k_ref[rows(r), :] * (1.0 / _TAU))
        k_ref[rows(r), :] = e
        s = jnp.sum(e, axis=1, keepdims=True)
        s_ref[rows(r), :] = s
        return acc + jnp.sum(e * (1.0 / s), axis=0, keepdims=True)

    acc0 = jnp.zeros((1, n), dtype=jnp.float32)
    v = 1.0 / jax.lax.fori_loop(0, n_chunks, init_chunk, acc0)

    def fused_pass(_, v):
        def chunk(r, acc):
            kc = k_ref[rows(r), :]
            s = jnp.sum(kc * v, axis=1, keepdims=True)
            s_ref[rows(r), :] = s
            return acc + jnp.sum(kc * (1.0 / s), axis=0, keepdims=True)

        return 1.0 / jax.lax.fori_loop(0, n_chunks, chunk, acc0)

    v = jax.lax.fori_loop(0, _MAX_ITER - 1, fused_pass, v)

    def prod_chunk(r, _):
        u = 1.0 / s_ref[rows(r), :]
        k_ref[rows(r), :] = k_ref[rows(r), :] * u * v
        return 0

    jax.lax.fori_loop(0, n_chunks, prod_chunk, 0)

    store = pltpu.make_async_copy(k_ref, out_hbm.at[b], out_sem)
    store.start()
    store.wait()


@jax.jit
def kernel(t):
    b, n, m = t.shape
    return pl.pallas_call(
        _sinkhorn_kernel,
        grid=(b,),
        in_specs=[pl.BlockSpec(memory_space=pltpu.MemorySpace.HBM)],
        out_specs=pl.BlockSpec(memory_space=pltpu.MemorySpace.HBM),
        out_shape=jax.ShapeDtypeStruct((b, n, m), jnp.float32),
        scratch_shapes=[
            pltpu.VMEM((n, m), jnp.float32),
            pltpu.VMEM((n, 1), jnp.float32),
            pltpu.SemaphoreType.DMA,
            pltpu.SemaphoreType.DMA,
        ],
    )(t)


# speedup vs baseline: 1.7245x; 1.1929x over previous
import jax
import jax.numpy as jnp
from jax.experimental import pallas as pl
from jax.experimental.pallas import tpu as pltpu

_MAX_ITER = 10
_TAU = 1.0
_CHUNK = 128


def _sinkhorn_kernel(t_hbm, out_hbm, k0_ref, k1_ref, s_ref,
                     in_sem0, in_sem1, out_sem0, out_sem1):
    b = pl.program_id(0)
    nb = pl.num_programs(0)
    n = k0_ref.shape[0]
    n_chunks = n // _CHUNK

    def rows(r):
        return pl.ds(r * _CHUNK, _CHUNK)

    def step(k_ref, oth_ref, in_sem, in_sem_oth, out_sem, out_sem_oth):
        @pl.when(b == 0)
        def _():
            pltpu.make_async_copy(t_hbm.at[b], k_ref, in_sem).start()

        pltpu.make_async_copy(t_hbm.at[b], k_ref, in_sem).wait()

        acc0 = jnp.zeros((1, n), dtype=jnp.float32)

        def init_chunk(r, acc):
            e = jnp.exp(k_ref[rows(r), :] * (1.0 / _TAU))
            k_ref[rows(r), :] = e
            s = jnp.sum(e, axis=1, keepdims=True)
            s_ref[rows(r), :] = s
            return acc + jnp.sum(e * (1.0 / s), axis=0, keepdims=True)

        v = 1.0 / jax.lax.fori_loop(0, n_chunks, init_chunk, acc0)

        @pl.when(b + 1 < nb)
        def _():
            @pl.when(b >= 1)
            def _():
                pltpu.make_async_copy(oth_ref, out_hbm.at[b - 1],
                                      out_sem_oth).wait()

            pltpu.make_async_copy(t_hbm.at[b + 1], oth_ref, in_sem_oth).start()

        def fused_pass(_, v):
            def chunk(r, acc):
                kc = k_ref[rows(r), :]
                s = jnp.sum(kc * v, axis=1, keepdims=True)
                s_ref[rows(r), :] = s
                return acc + jnp.sum(kc * (1.0 / s), axis=0, keepdims=True)

            return 1.0 / jax.lax.fori_loop(0, n_chunks, chunk, acc0)

        v = jax.lax.fori_loop(0, _MAX_ITER - 1, fused_pass, v)

        def prod_chunk(r, _):
            u = 1.0 / s_ref[rows(r), :]
            k_ref[rows(r), :] = k_ref[rows(r), :] * u * v
            return 0

        jax.lax.fori_loop(0, n_chunks, prod_chunk, 0)

        pltpu.make_async_copy(k_ref, out_hbm.at[b], out_sem).start()

        @pl.when(b == nb - 1)
        def _():
            @pl.when(b >= 1)
            def _():
                pltpu.make_async_copy(oth_ref, out_hbm.at[b - 1],
                                      out_sem_oth).wait()

            pltpu.make_async_copy(k_ref, out_hbm.at[b], out_sem).wait()

    even = jax.lax.rem(b, 2) == 0

    @pl.when(even)
    def _():
        step(k0_ref, k1_ref, in_sem0, in_sem1, out_sem0, out_sem1)

    @pl.when(jnp.logical_not(even))
    def _():
        step(k1_ref, k0_ref, in_sem1, in_sem0, out_sem1, out_sem0)


@jax.jit
def kernel(t):
    b, n, m = t.shape
    return pl.pallas_call(
        _sinkhorn_kernel,
        grid=(b,),
        in_specs=[pl.BlockSpec(memory_space=pltpu.MemorySpace.HBM)],
        out_specs=pl.BlockSpec(memory_space=pltpu.MemorySpace.HBM),
        out_shape=jax.ShapeDtypeStruct((b, n, m), jnp.float32),
        scratch_shapes=[
            pltpu.VMEM((n, m), jnp.float32),
            pltpu.VMEM((n, m), jnp.float32),
            pltpu.VMEM((n, 1), jnp.float32),
            pltpu.SemaphoreType.DMA,
            pltpu.SemaphoreType.DMA,
            pltpu.SemaphoreType.DMA,
            pltpu.SemaphoreType.DMA,
        ],
    )(t)


# speedup vs baseline: 2.1800x; 1.2641x over previous
import jax
import jax.numpy as jnp
from jax.experimental import pallas as pl
from jax.experimental.pallas import tpu as pltpu

_MAX_ITER = 10
_TAU = 1.0
_CHUNK = 256


def _sinkhorn_kernel(t_hbm, out_hbm, k0_ref, k1_ref, s_ref,
                     in_sem0, in_sem1, out_sem0, out_sem1):
    b = pl.program_id(0)
    nb = pl.num_programs(0)
    n = k0_ref.shape[0]
    n_chunks = n // _CHUNK

    def rows(r):
        return pl.ds(r * _CHUNK, _CHUNK)

    def step(k_ref, oth_ref, in_sem, in_sem_oth, out_sem, out_sem_oth):
        @pl.when(b == 0)
        def _():
            pltpu.make_async_copy(t_hbm.at[b], k_ref, in_sem).start()

        pltpu.make_async_copy(t_hbm.at[b], k_ref, in_sem).wait()

        acc0 = jnp.zeros((1, n), dtype=jnp.float32)

        def init_chunk(r, acc):
            e = jnp.exp(k_ref[rows(r), :] * (1.0 / _TAU))
            k_ref[rows(r), :] = e
            s = jnp.sum(e, axis=1, keepdims=True)
            s_ref[rows(r), :] = s
            return acc + jnp.sum(e * (1.0 / s), axis=0, keepdims=True)

        v = 1.0 / jax.lax.fori_loop(0, n_chunks, init_chunk, acc0)

        @pl.when(b + 1 < nb)
        def _():
            @pl.when(b >= 1)
            def _():
                pltpu.make_async_copy(oth_ref, out_hbm.at[b - 1],
                                      out_sem_oth).wait()

            pltpu.make_async_copy(t_hbm.at[b + 1], oth_ref, in_sem_oth).start()

        def fused_pass(_, v):
            def chunk(r, acc):
                kc = k_ref[rows(r), :]
                s = jnp.sum(kc * v, axis=1, keepdims=True)
                s_ref[rows(r), :] = s
                return acc + jnp.sum(kc * (1.0 / s), axis=0, keepdims=True)

            return 1.0 / jax.lax.fori_loop(0, n_chunks, chunk, acc0)

        v = jax.lax.fori_loop(0, _MAX_ITER - 1, fused_pass, v)

        def prod_chunk(r, _):
            u = 1.0 / s_ref[rows(r), :]
            k_ref[rows(r), :] = k_ref[rows(r), :] * u * v
            return 0

        jax.lax.fori_loop(0, n_chunks, prod_chunk, 0)

        pltpu.make_async_copy(k_ref, out_hbm.at[b], out_sem).start()

        @pl.when(b == nb - 1)
        def _():
            @pl.when(b >= 1)
            def _():
                pltpu.make_async_copy(oth_ref, out_hbm.at[b - 1],
                                      out_sem_oth).wait()

            pltpu.make_async_copy(k_ref, out_hbm.at[b], out_sem).wait()

    even = jax.lax.rem(b, 2) == 0

    @pl.when(even)
    def _():
        step(k0_ref, k1_ref, in_sem0, in_sem1, out_sem0, out_sem1)

    @pl.when(jnp.logical_not(even))
    def _():
        step(k1_ref, k0_ref, in_sem1, in_sem0, out_sem1, out_sem0)


@jax.jit
def kernel(t):
    b, n, m = t.shape
    return pl.pallas_call(
        _sinkhorn_kernel,
        grid=(b,),
        in_specs=[pl.BlockSpec(memory_space=pltpu.MemorySpace.HBM)],
        out_specs=pl.BlockSpec(memory_space=pltpu.MemorySpace.HBM),
        out_shape=jax.ShapeDtypeStruct((b, n, m), jnp.float32),
        scratch_shapes=[
            pltpu.VMEM((n, m), jnp.float32),
            pltpu.VMEM((n, m), jnp.float32),
            pltpu.VMEM((n, 1), jnp.float32),
            pltpu.SemaphoreType.DMA,
            pltpu.SemaphoreType.DMA,
            pltpu.SemaphoreType.DMA,
            pltpu.SemaphoreType.DMA,
        ],
    )(t)


# speedup vs baseline: 2.2956x; 1.0530x over previous
import jax
import jax.numpy as jnp
from jax.experimental import pallas as pl
from jax.experimental.pallas import tpu as pltpu

_MAX_ITER = 10
_TAU = 1.0
_CHUNK = 512


def _sinkhorn_kernel(t_hbm, out_hbm, k0_ref, k1_ref, s_ref,
                     in_sem0, in_sem1, out_sem0, out_sem1):
    b = pl.program_id(0)
    nb = pl.num_programs(0)
    n = k0_ref.shape[0]
    n_chunks = n // _CHUNK

    def rows(r):
        return pl.ds(r * _CHUNK, _CHUNK)

    def step(k_ref, oth_ref, in_sem, in_sem_oth, out_sem, out_sem_oth):
        @pl.when(b == 0)
        def _():
            pltpu.make_async_copy(t_hbm.at[b], k_ref, in_sem).start()

        pltpu.make_async_copy(t_hbm.at[b], k_ref, in_sem).wait()

        acc0 = jnp.zeros((1, n), dtype=jnp.float32)

        def init_chunk(r, acc):
            e = jnp.exp(k_ref[rows(r), :] * (1.0 / _TAU))
            k_ref[rows(r), :] = e
            s = jnp.sum(e, axis=1, keepdims=True)
            s_ref[rows(r), :] = s
            return acc + jnp.sum(e * (1.0 / s), axis=0, keepdims=True)

        v = 1.0 / jax.lax.fori_loop(0, n_chunks, init_chunk, acc0)

        @pl.when(b + 1 < nb)
        def _():
            @pl.when(b >= 1)
            def _():
                pltpu.make_async_copy(oth_ref, out_hbm.at[b - 1],
                                      out_sem_oth).wait()

            pltpu.make_async_copy(t_hbm.at[b + 1], oth_ref, in_sem_oth).start()

        def fused_pass(_, v):
            def chunk(r, acc):
                kc = k_ref[rows(r), :]
                s = jnp.sum(kc * v, axis=1, keepdims=True)
                s_ref[rows(r), :] = s
                return acc + jnp.sum(kc * (1.0 / s), axis=0, keepdims=True)

            return 1.0 / jax.lax.fori_loop(0, n_chunks, chunk, acc0)

        v = jax.lax.fori_loop(0, _MAX_ITER - 1, fused_pass, v)

        def prod_chunk(r, _):
            u = 1.0 / s_ref[rows(r), :]
            k_ref[rows(r), :] = k_ref[rows(r), :] * u * v
            return 0

        jax.lax.fori_loop(0, n_chunks, prod_chunk, 0)

        pltpu.make_async_copy(k_ref, out_hbm.at[b], out_sem).start()

        @pl.when(b == nb - 1)
        def _():
            @pl.when(b >= 1)
            def _():
                pltpu.make_async_copy(oth_ref, out_hbm.at[b - 1],
                                      out_sem_oth).wait()

            pltpu.make_async_copy(k_ref, out_hbm.at[b], out_sem).wait()

    even = jax.lax.rem(b, 2) == 0

    @pl.when(even)
    def _():
        step(k0_ref, k1_ref, in_sem0, in_sem1, out_sem0, out_sem1)

    @pl.when(jnp.logical_not(even))
    def _():
        step(k1_ref, k0_ref, in_sem1, in_sem0, out_sem1, out_sem0)


@jax.jit
def kernel(t):
    b, n, m = t.shape
    return pl.pallas_call(
        _sinkhorn_kernel,
        grid=(b,),
        in_specs=[pl.BlockSpec(memory_space=pltpu.MemorySpace.HBM)],
        out_specs=pl.BlockSpec(memory_space=pltpu.MemorySpace.HBM),
        out_shape=jax.ShapeDtypeStruct((b, n, m), jnp.float32),
        scratch_shapes=[
            pltpu.VMEM((n, m), jnp.float32),
            pltpu.VMEM((n, m), jnp.float32),
            pltpu.VMEM((n, 1), jnp.float32),
            pltpu.SemaphoreType.DMA,
            pltpu.SemaphoreType.DMA,
            pltpu.SemaphoreType.DMA,
            pltpu.SemaphoreType.DMA,
        ],
    )(t)
